# Initial kernel scaffold; baseline (speedup 1.0000x reference)
import jax
import jax.numpy as jnp
from jax.experimental import pallas as pl


def kernel(x, edge_index, edge_attr, Wl1, bl1, Wr1, br1, We1, att1, b1, Wl2, bl2, Wr2, br2, We2, att2, b2):
    raise NotImplementedError("write your pallas kernel here")


# speedup vs baseline: 4.1201x; 4.1201x over previous
import dataclasses
import functools

import jax
import jax.numpy as jnp
from jax import lax
from jax.experimental import pallas as pl
from jax.experimental.pallas import tpu as pltpu
from jax.experimental.pallas import tpu_sc as plsc

SCC = 1
NS = 16
L = 16
NEG_SLOPE = 0.2


def _tc_pre(x, Wl, bl, Wr, br):
    N, _ = x.shape
    C = Wl.shape[1]

    def body(x_ref, wl_ref, bl_ref, wr_ref, br_ref, xl_ref, xr_ref):
        xv = x_ref[...]
        xl_ref[...] = (
            jnp.dot(xv, wl_ref[...], preferred_element_type=jnp.float32)
            + bl_ref[...]
        )
        xr_ref[...] = (
            jnp.dot(xv, wr_ref[...], preferred_element_type=jnp.float32)
            + br_ref[...]
        )

    return pl.pallas_call(
        body,
        out_shape=(
            jax.ShapeDtypeStruct((N, C), jnp.float32),
            jax.ShapeDtypeStruct((N, C), jnp.float32),
        ),
    )(x, Wl, bl.reshape(1, C), Wr, br.reshape(1, C))


def _tc_mid(num, den, b1, Wl, bl, Wr, br, N):
    C = Wl.shape[0]

    def body(n_ref, d_ref, b1_ref, wl_ref, bl_ref, wr_ref, br_ref,
             xl_ref, xr_ref):
        nv = n_ref[0:N, :]
        ones = jnp.ones((d_ref.shape[0], 1), jnp.float32)
        dv = lax.dot_general(
            d_ref[...], ones, (((0,), (0,)), ((), ())),
            preferred_element_type=jnp.float32)[0:N, :]
        h = jnp.maximum(nv / (dv + 1e-16) + b1_ref[...], 0.0)
        xl_ref[...] = (
            jnp.dot(h, wl_ref[...], preferred_element_type=jnp.float32)
            + bl_ref[...]
        )
        xr_ref[...] = (
            jnp.dot(h, wr_ref[...], preferred_element_type=jnp.float32)
            + br_ref[...]
        )

    return pl.pallas_call(
        body,
        out_shape=(
            jax.ShapeDtypeStruct((N, C), jnp.float32),
            jax.ShapeDtypeStruct((N, C), jnp.float32),
        ),
    )(num, den, b1.reshape(1, C), Wl, bl.reshape(1, C),
      Wr, br.reshape(1, C))


def _tc_post(num, den, b2, C, N):
    def body(n_ref, d_ref, b2_ref, o_ref):
        nv = n_ref[0:N, :]
        ones = jnp.ones((d_ref.shape[0], 1), jnp.float32)
        dv = lax.dot_general(
            d_ref[...], ones, (((0,), (0,)), ((), ())),
            preferred_element_type=jnp.float32)[0:N, :]
        o_ref[...] = nv / (dv + 1e-16) + b2_ref[...]

    return pl.pallas_call(
        body,
        out_shape=jax.ShapeDtypeStruct((N, C), jnp.float32),
    )(num, den, b2.reshape(1, C))


@functools.lru_cache(maxsize=None)
def _make_sc_edge_pass(N, C, NW, NGRP, NBS, B):
    EPT = NGRP * NBS * B
    NPS = -(-(N // NS) // L) * L
    NPAD = NPS * NS
    NCH = C // L
    mesh = plsc.VectorSubcoreMesh(
        core_axis_name="c", subcore_axis_name="s", num_cores=SCC)
    cp = pltpu.CompilerParams()
    if "needs_layout_passes" in pltpu.CompilerParams.__dataclass_fields__:
        cp = dataclasses.replace(cp, needs_layout_passes=False)

    @functools.partial(
        pl.kernel,
        out_type=(
            jax.ShapeDtypeStruct((SCC * NPAD, C), jnp.float32),
            jax.ShapeDtypeStruct((SCC * NS, NPAD), jnp.float32),
        ),
        mesh=mesh,
        compiler_params=cp,
        scratch_types=[
            pltpu.VMEM((NBS, B), jnp.int32),
            pltpu.VMEM((NBS, B), jnp.int32),
            pltpu.VMEM((NBS * B,), jnp.float32),
            pltpu.VMEM((2, C), jnp.float32),
            pltpu.VMEM((B, C), jnp.float32),
            pltpu.VMEM((B, C), jnp.float32),
            pltpu.VMEM((NPAD,), jnp.float32),
            pltpu.VMEM_SHARED((NPAD, C), jnp.float32),
            pltpu.SemaphoreType.DMA,
            pltpu.SemaphoreType.DMA,
        ],
    )
    def k(xl_hbm, xr_hbm, src_hbm, dst_hbm, ea_hbm, cst_hbm,
          num_hbm, den_hbm,
          sidx, didx, eas, cv, xlr, xrr, dent,
          accs, sem1, sem2):
        cid = lax.axis_index("c")
        sid = lax.axis_index("s")
        wid = cid * NS + sid
        pltpu.sync_copy(cst_hbm, cv)

        zv = jnp.zeros((L,), jnp.float32)

        @pl.loop(0, NPAD // L)
        def _(j):
            dent[pl.ds(j * L, L)] = zv

        @pl.loop(0, B)
        def _(r):
            for c in range(NCH):
                xlr[r, pl.ds(c * L, L)] = zv

        z0 = sid * NPS
        for j in range(NPS // B):
            pltpu.sync_copy(xlr, accs.at[pl.ds(z0 + j * B, B)])
        plsc.subcore_barrier()

        wec = [cv[0, pl.ds(c * L, L)] for c in range(NCH)]
        attc = [cv[1, pl.ds(c * L, L)] for c in range(NCH)]
        lane0 = lax.iota(jnp.int32, L) == 0

        @pl.loop(0, NGRP)
        def _(g):
            pltpu.sync_copy(src_hbm.at[wid, g], sidx)
            pltpu.sync_copy(dst_hbm.at[wid, g], didx)
            pltpu.sync_copy(
                ea_hbm.at[pl.ds(wid * EPT + g * (NBS * B), NBS * B)], eas)

            for j in range(NBS):
                pltpu.async_copy(xl_hbm.at[sidx.at[j]], xlr, sem1).wait()
                pltpu.async_copy(xr_hbm.at[didx.at[j]], xrr, sem2).wait()

                @pl.loop(0, B)
                def _(b):
                    eab = plsc.load_gather(
                        eas, [lax.broadcast(j * B + b, (L,))])
                    acc = jnp.zeros((L,), jnp.float32)
                    xs = []
                    for c in range(NCH):
                        xlc = xlr[b, pl.ds(c * L, L)]
                        v = xlc + xrr[b, pl.ds(c * L, L)] + eab * wec[c]
                        v = jnp.maximum(v, NEG_SLOPE * v)
                        acc = acc + v * attc[c]
                        xs.append(xlc)
                    ex = jnp.exp(lax.broadcast(jnp.sum(acc), (L,)))
                    for c in range(NCH):
                        xlr[b, pl.ds(c * L, L)] = xs[c] * ex
                    dstb = plsc.load_gather(didx, [lax.broadcast(j, (L,)),
                                                   lax.broadcast(b, (L,))])
                    plsc.addupdate_scatter(dent, [dstb], ex, mask=lane0)

                pltpu.sync_copy(xlr, accs.at[didx.at[j]], add=True)

        pltpu.sync_copy(dent, den_hbm.at[cid * NS + sid])
        plsc.subcore_barrier()
        base = cid * NPAD + sid * NPS
        pltpu.sync_copy(accs.at[pl.ds(sid * NPS, NPS)],
                        num_hbm.at[pl.ds(base, NPS)])

    return k


def _sc_edge_pass(xl, xr, src4, dst4, ea, consts, B):
    N, C = xl.shape
    NW, NGRP, NBS, _ = src4.shape
    k = _make_sc_edge_pass(N, C, NW, NGRP, NBS, B)
    return k(xl, xr, src4, dst4, ea, consts)


def kernel(x, edge_index, edge_attr,
           Wl1, bl1, Wr1, br1, We1, att1, b1,
           Wl2, bl2, Wr2, br2, We2, att2, b2):
    N, D = x.shape
    C = Wl1.shape[1]
    E = edge_index.shape[1]
    NW = SCC * NS
    EPT = E // NW
    B = 80
    assert E % NW == 0 and EPT % B == 0 and N % NS == 0

    NBS = 10
    NGRP = EPT // (NBS * B)
    assert NGRP * NBS * B == EPT
    src = edge_index[0].astype(jnp.int32)
    dst = edge_index[1].astype(jnp.int32)
    ea = edge_attr.reshape(E)
    src3 = src.reshape(NW, NGRP, NBS, B)
    dst3 = dst.reshape(NW, NGRP, NBS, B)
    consts1 = jnp.concatenate(
        [We1.reshape(1, C), att1.reshape(1, C)], axis=0)
    consts2 = jnp.concatenate(
        [We2.reshape(1, C), att2.reshape(1, C)], axis=0)

    xl1, xr1 = _tc_pre(x, Wl1, bl1, Wr1, br1)
    num1, den1 = _sc_edge_pass(xl1, xr1, src3, dst3, ea, consts1, B)
    xl2, xr2 = _tc_mid(num1, den1, b1, Wl2, bl2, Wr2, br2, N)
    num2, den2 = _sc_edge_pass(xl2, xr2, src3, dst3, ea, consts2, B)
    return _tc_post(num2, den2, b2, C, N)


# speedup vs baseline: 7.7846x; 1.8894x over previous
import dataclasses
import functools

import jax
import jax.numpy as jnp
from jax import lax
from jax.experimental import pallas as pl
from jax.experimental.pallas import tpu as pltpu
from jax.experimental.pallas import tpu_sc as plsc

SCC = 2
NS = 16
L = 16
NEG_SLOPE = 0.2


def _tc_pre(x, Wl, bl, Wr, br):
    N, _ = x.shape
    C = Wl.shape[1]

    def body(x_ref, wl_ref, bl_ref, wr_ref, br_ref, xl_ref, xr_ref):
        xv = x_ref[...]
        xl_ref[...] = (
            jnp.dot(xv, wl_ref[...], preferred_element_type=jnp.float32)
            + bl_ref[...]
        )
        xr_ref[...] = (
            jnp.dot(xv, wr_ref[...], preferred_element_type=jnp.float32)
            + br_ref[...]
        )

    return pl.pallas_call(
        body,
        out_shape=(
            jax.ShapeDtypeStruct((N, C), jnp.float32),
            jax.ShapeDtypeStruct((N, C), jnp.float32),
        ),
    )(x, Wl, bl.reshape(1, C), Wr, br.reshape(1, C))


def _tc_mid(num, den, b1, Wl, bl, Wr, br, N):
    C = Wl.shape[0]

    def body(n_ref, d_ref, b1_ref, wl_ref, bl_ref, wr_ref, br_ref,
             xl_ref, xr_ref):
        P = n_ref.shape[0] // SCC
        nv = n_ref[0:N, :]
        for p in range(1, SCC):
            nv = nv + n_ref[p * P : p * P + N, :]
        ones = jnp.ones((d_ref.shape[0], 1), jnp.float32)
        dv = lax.dot_general(
            d_ref[...], ones, (((0,), (0,)), ((), ())),
            preferred_element_type=jnp.float32)[0:N, :]
        h = jnp.maximum(nv / (dv + 1e-16) + b1_ref[...], 0.0)
        xl_ref[...] = (
            jnp.dot(h, wl_ref[...], preferred_element_type=jnp.float32)
            + bl_ref[...]
        )
        xr_ref[...] = (
            jnp.dot(h, wr_ref[...], preferred_element_type=jnp.float32)
            + br_ref[...]
        )

    return pl.pallas_call(
        body,
        out_shape=(
            jax.ShapeDtypeStruct((N, C), jnp.float32),
            jax.ShapeDtypeStruct((N, C), jnp.float32),
        ),
    )(num, den, b1.reshape(1, C), Wl, bl.reshape(1, C),
      Wr, br.reshape(1, C))


def _tc_post(num, den, b2, C, N):
    def body(n_ref, d_ref, b2_ref, o_ref):
        P = n_ref.shape[0] // SCC
        nv = n_ref[0:N, :]
        for p in range(1, SCC):
            nv = nv + n_ref[p * P : p * P + N, :]
        ones = jnp.ones((d_ref.shape[0], 1), jnp.float32)
        dv = lax.dot_general(
            d_ref[...], ones, (((0,), (0,)), ((), ())),
            preferred_element_type=jnp.float32)[0:N, :]
        o_ref[...] = nv / (dv + 1e-16) + b2_ref[...]

    return pl.pallas_call(
        body,
        out_shape=jax.ShapeDtypeStruct((N, C), jnp.float32),
    )(num, den, b2.reshape(1, C))


@functools.lru_cache(maxsize=None)
def _make_sc_edge_pass(N, C, NW, NGRP, NBS, B):
    EPT = NGRP * NBS * B
    NPS = -(-(N // NS) // L) * L
    NPAD = NPS * NS
    NCH = C // L
    mesh = plsc.VectorSubcoreMesh(
        core_axis_name="c", subcore_axis_name="s", num_cores=SCC)
    cp = pltpu.CompilerParams()
    if "needs_layout_passes" in pltpu.CompilerParams.__dataclass_fields__:
        cp = dataclasses.replace(cp, needs_layout_passes=False)

    @functools.partial(
        pl.kernel,
        out_type=(
            jax.ShapeDtypeStruct((SCC * NPAD, C), jnp.float32),
            jax.ShapeDtypeStruct((SCC * NS, NPAD), jnp.float32),
        ),
        mesh=mesh,
        compiler_params=cp,
        scratch_types=[
            pltpu.VMEM((NBS, B), jnp.int32),
            pltpu.VMEM((NBS, B), jnp.int32),
            pltpu.VMEM((NBS * B,), jnp.float32),
            pltpu.VMEM((2, C), jnp.float32),
            pltpu.VMEM((B, C), jnp.float32),
            pltpu.VMEM((B, C), jnp.float32),
            pltpu.VMEM((NPAD,), jnp.float32),
            pltpu.VMEM_SHARED((NPAD, C), jnp.float32),
            pltpu.SemaphoreType.DMA,
            pltpu.SemaphoreType.DMA,
        ],
    )
    def k(xl_hbm, xr_hbm, src_hbm, dst_hbm, ea_hbm, cst_hbm,
          num_hbm, den_hbm,
          sidx, didx, eas, cv, xlr, xrr, dent,
          accs, sem1, sem2):
        cid = lax.axis_index("c")
        sid = lax.axis_index("s")
        wid = cid * NS + sid
        pltpu.sync_copy(cst_hbm, cv)

        zv = jnp.zeros((L,), jnp.float32)

        @pl.loop(0, NPAD // L)
        def _(j):
            dent[pl.ds(j * L, L)] = zv

        @pl.loop(0, B)
        def _(r):
            for c in range(NCH):
                xlr[r, pl.ds(c * L, L)] = zv

        z0 = sid * NPS
        for j in range(NPS // B):
            pltpu.sync_copy(xlr, accs.at[pl.ds(z0 + j * B, B)])
        plsc.subcore_barrier()

        wec = [cv[0, pl.ds(c * L, L)] for c in range(NCH)]
        attc = [cv[1, pl.ds(c * L, L)] for c in range(NCH)]
        lane0 = lax.iota(jnp.int32, L) == 0

        @pl.loop(0, NGRP)
        def _(g):
            pltpu.sync_copy(src_hbm.at[wid, g], sidx)
            pltpu.sync_copy(dst_hbm.at[wid, g], didx)
            pltpu.sync_copy(
                ea_hbm.at[pl.ds(wid * EPT + g * (NBS * B), NBS * B)], eas)

            for j in range(NBS):
                pltpu.async_copy(xl_hbm.at[sidx.at[j]], xlr, sem1).wait()
                pltpu.async_copy(xr_hbm.at[didx.at[j]], xrr, sem2).wait()

                @pl.loop(0, B)
                def _(b):
                    eab = plsc.load_gather(
                        eas, [lax.broadcast(j * B + b, (L,))])
                    acc = jnp.zeros((L,), jnp.float32)
                    xs = []
                    for c in range(NCH):
                        xlc = xlr[b, pl.ds(c * L, L)]
                        v = xlc + xrr[b, pl.ds(c * L, L)] + eab * wec[c]
                        v = jnp.maximum(v, NEG_SLOPE * v)
                        acc = acc + v * attc[c]
                        xs.append(xlc)
                    ex = jnp.exp(lax.broadcast(jnp.sum(acc), (L,)))
                    for c in range(NCH):
                        xlr[b, pl.ds(c * L, L)] = xs[c] * ex
                    dstb = plsc.load_gather(didx, [lax.broadcast(j, (L,)),
                                                   lax.broadcast(b, (L,))])
                    plsc.addupdate_scatter(dent, [dstb], ex, mask=lane0)

                pltpu.sync_copy(xlr, accs.at[didx.at[j]], add=True)

        pltpu.sync_copy(dent, den_hbm.at[cid * NS + sid])
        plsc.subcore_barrier()
        base = cid * NPAD + sid * NPS
        pltpu.sync_copy(accs.at[pl.ds(sid * NPS, NPS)],
                        num_hbm.at[pl.ds(base, NPS)])

    return k


def _sc_edge_pass(xl, xr, src4, dst4, ea, consts, B):
    N, C = xl.shape
    NW, NGRP, NBS, _ = src4.shape
    k = _make_sc_edge_pass(N, C, NW, NGRP, NBS, B)
    return k(xl, xr, src4, dst4, ea, consts)


def kernel(x, edge_index, edge_attr,
           Wl1, bl1, Wr1, br1, We1, att1, b1,
           Wl2, bl2, Wr2, br2, We2, att2, b2):
    N, D = x.shape
    C = Wl1.shape[1]
    E = edge_index.shape[1]
    NW = SCC * NS
    EPT = E // NW
    B = 80
    assert E % NW == 0 and EPT % B == 0 and N % NS == 0

    NBS = next(n for n in range(10, 0, -1) if EPT % (n * B) == 0)
    NGRP = EPT // (NBS * B)
    assert NGRP * NBS * B == EPT
    src = edge_index[0].astype(jnp.int32)
    dst = edge_index[1].astype(jnp.int32)
    ea = edge_attr.reshape(E)
    src3 = src.reshape(NW, NGRP, NBS, B)
    dst3 = dst.reshape(NW, NGRP, NBS, B)
    consts1 = jnp.concatenate(
        [We1.reshape(1, C), att1.reshape(1, C)], axis=0)
    consts2 = jnp.concatenate(
        [We2.reshape(1, C), att2.reshape(1, C)], axis=0)

    xl1, xr1 = _tc_pre(x, Wl1, bl1, Wr1, br1)
    num1, den1 = _sc_edge_pass(xl1, xr1, src3, dst3, ea, consts1, B)
    xl2, xr2 = _tc_mid(num1, den1, b1, Wl2, bl2, Wr2, br2, N)
    num2, den2 = _sc_edge_pass(xl2, xr2, src3, dst3, ea, consts2, B)
    return _tc_post(num2, den2, b2, C, N)


# speedup vs baseline: 10.4244x; 1.3391x over previous
import dataclasses
import functools

import jax
import jax.numpy as jnp
from jax import lax
from jax.experimental import pallas as pl
from jax.experimental.pallas import tpu as pltpu
from jax.experimental.pallas import tpu_sc as plsc

SCC = 2
NS = 16
L = 16
NEG_SLOPE = 0.2


def _tc_pre(x, Wl, bl, Wr, br):
    N, _ = x.shape
    C = Wl.shape[1]

    def body(x_ref, wl_ref, bl_ref, wr_ref, br_ref, xl_ref, xr_ref):
        xv = x_ref[...]
        xl_ref[...] = (
            jnp.dot(xv, wl_ref[...], preferred_element_type=jnp.float32)
            + bl_ref[...]
        )
        xr_ref[...] = (
            jnp.dot(xv, wr_ref[...], preferred_element_type=jnp.float32)
            + br_ref[...]
        )

    return pl.pallas_call(
        body,
        out_shape=(
            jax.ShapeDtypeStruct((N, C), jnp.float32),
            jax.ShapeDtypeStruct((N, C), jnp.float32),
        ),
    )(x, Wl, bl.reshape(1, C), Wr, br.reshape(1, C))


def _tc_mid(num, den, b1, Wl, bl, Wr, br, N):
    C = Wl.shape[0]

    def body(n_ref, d_ref, b1_ref, wl_ref, bl_ref, wr_ref, br_ref,
             xl_ref, xr_ref):
        P = n_ref.shape[0] // SCC
        nv = n_ref[0:N, :]
        for p in range(1, SCC):
            nv = nv + n_ref[p * P : p * P + N, :]
        ones = jnp.ones((d_ref.shape[0], 1), jnp.float32)
        dv = lax.dot_general(
            d_ref[...], ones, (((0,), (0,)), ((), ())),
            preferred_element_type=jnp.float32)[0:N, :]
        h = jnp.maximum(nv / (dv + 1e-16) + b1_ref[...], 0.0)
        xl_ref[...] = (
            jnp.dot(h, wl_ref[...], preferred_element_type=jnp.float32)
            + bl_ref[...]
        )
        xr_ref[...] = (
            jnp.dot(h, wr_ref[...], preferred_element_type=jnp.float32)
            + br_ref[...]
        )

    return pl.pallas_call(
        body,
        out_shape=(
            jax.ShapeDtypeStruct((N, C), jnp.float32),
            jax.ShapeDtypeStruct((N, C), jnp.float32),
        ),
    )(num, den, b1.reshape(1, C), Wl, bl.reshape(1, C),
      Wr, br.reshape(1, C))


def _tc_post(num, den, b2, C, N):
    def body(n_ref, d_ref, b2_ref, o_ref):
        P = n_ref.shape[0] // SCC
        nv = n_ref[0:N, :]
        for p in range(1, SCC):
            nv = nv + n_ref[p * P : p * P + N, :]
        ones = jnp.ones((d_ref.shape[0], 1), jnp.float32)
        dv = lax.dot_general(
            d_ref[...], ones, (((0,), (0,)), ((), ())),
            preferred_element_type=jnp.float32)[0:N, :]
        o_ref[...] = nv / (dv + 1e-16) + b2_ref[...]

    return pl.pallas_call(
        body,
        out_shape=jax.ShapeDtypeStruct((N, C), jnp.float32),
    )(num, den, b2.reshape(1, C))


@functools.lru_cache(maxsize=None)
def _make_sc_edge_pass(N, C, NW, NGRP, NBS, B):
    EPT = NGRP * NBS * B
    NPS = -(-(N // NS) // L) * L
    NPAD = NPS * NS
    NCH = C // L
    mesh = plsc.VectorSubcoreMesh(
        core_axis_name="c", subcore_axis_name="s", num_cores=SCC)
    cp = pltpu.CompilerParams()
    if "needs_layout_passes" in pltpu.CompilerParams.__dataclass_fields__:
        cp = dataclasses.replace(cp, needs_layout_passes=False)

    @functools.partial(
        pl.kernel,
        out_type=(
            jax.ShapeDtypeStruct((SCC * NPAD, C), jnp.float32),
            jax.ShapeDtypeStruct((SCC * NS, NPAD), jnp.float32),
        ),
        mesh=mesh,
        compiler_params=cp,
        scratch_types=[
            pltpu.VMEM((NBS, B), jnp.int32),
            pltpu.VMEM((NBS, B), jnp.int32),
            pltpu.VMEM((NBS * B,), jnp.float32),
            pltpu.VMEM((2, C), jnp.float32),
            pltpu.VMEM((B, C), jnp.float32),
            pltpu.VMEM((B, C), jnp.float32),
            pltpu.VMEM((B, C), jnp.float32),
            pltpu.VMEM((B, C), jnp.float32),
            pltpu.VMEM((NPAD,), jnp.float32),
            pltpu.VMEM_SHARED((NPAD, C), jnp.float32),
            pltpu.SemaphoreType.DMA,
            pltpu.SemaphoreType.DMA,
            pltpu.SemaphoreType.DMA,
            pltpu.SemaphoreType.DMA,
        ],
    )
    def k(xl_hbm, xr_hbm, src_hbm, dst_hbm, ea_hbm, cst_hbm,
          num_hbm, den_hbm,
          sidx, didx, eas, cv, xla, xlb, xra, xrb, dent,
          accs, gsl, gsr, ssa, ssb):
        xlbuf = (xla, xlb)
        xrbuf = (xra, xrb)
        ssem = (ssa, ssb)
        cid = lax.axis_index("c")
        sid = lax.axis_index("s")
        wid = cid * NS + sid
        pltpu.sync_copy(cst_hbm, cv)

        zv = jnp.zeros((L,), jnp.float32)

        @pl.loop(0, NPAD // L)
        def _(j):
            dent[pl.ds(j * L, L)] = zv

        @pl.loop(0, B)
        def _(r):
            for c in range(NCH):
                xla[r, pl.ds(c * L, L)] = zv

        z0 = sid * NPS
        for j in range(NPS // B):
            pltpu.sync_copy(xla, accs.at[pl.ds(z0 + j * B, B)])
        plsc.subcore_barrier()

        wec = [cv[0, pl.ds(c * L, L)] for c in range(NCH)]
        attc = [cv[1, pl.ds(c * L, L)] for c in range(NCH)]
        lane0 = lax.iota(jnp.int32, L) == 0

        @pl.loop(0, NGRP)
        def _(g):
            pltpu.sync_copy(src_hbm.at[wid, g], sidx)
            pltpu.sync_copy(dst_hbm.at[wid, g], didx)
            pltpu.sync_copy(
                ea_hbm.at[pl.ds(wid * EPT + g * (NBS * B), NBS * B)], eas)

            pltpu.async_copy(xl_hbm.at[sidx.at[0]], xlbuf[0], gsl)
            pltpu.async_copy(xr_hbm.at[didx.at[0]], xrbuf[0], gsr)
            for j in range(NBS):
                p = j % 2
                q = 1 - p
                xlr = xlbuf[p]
                xrr = xrbuf[p]
                if j < NBS - 1:
                    if j >= 1:
                        pltpu.make_async_copy(
                            xlbuf[q], accs.at[didx.at[j - 1]],
                            ssem[q]).wait()
                    pltpu.async_copy(xl_hbm.at[sidx.at[j + 1]],
                                     xlbuf[q], gsl)
                    pltpu.async_copy(xr_hbm.at[didx.at[j + 1]],
                                     xrbuf[q], gsr)
                pltpu.make_async_copy(
                    xl_hbm.at[sidx.at[j]], xlr, gsl).wait()
                pltpu.make_async_copy(
                    xr_hbm.at[didx.at[j]], xrr, gsr).wait()

                @pl.loop(0, B)
                def _(b):
                    eab = plsc.load_gather(
                        eas, [lax.broadcast(j * B + b, (L,))])
                    acc = jnp.zeros((L,), jnp.float32)
                    xs = []
                    for c in range(NCH):
                        xlc = xlr[b, pl.ds(c * L, L)]
                        v = xlc + xrr[b, pl.ds(c * L, L)] + eab * wec[c]
                        v = jnp.maximum(v, NEG_SLOPE * v)
                        acc = acc + v * attc[c]
                        xs.append(xlc)
                    ex = jnp.exp(lax.broadcast(jnp.sum(acc), (L,)))
                    for c in range(NCH):
                        xlr[b, pl.ds(c * L, L)] = xs[c] * ex
                    dstb = plsc.load_gather(didx, [lax.broadcast(j, (L,)),
                                                   lax.broadcast(b, (L,))])
                    plsc.addupdate_scatter(dent, [dstb], ex, mask=lane0)

                pltpu.async_copy(xlr, accs.at[didx.at[j]], ssem[p],
                                 add=True)

            pltpu.make_async_copy(
                xlbuf[(NBS - 2) % 2], accs.at[didx.at[NBS - 2]],
                ssem[(NBS - 2) % 2]).wait()
            pltpu.make_async_copy(
                xlbuf[(NBS - 1) % 2], accs.at[didx.at[NBS - 1]],
                ssem[(NBS - 1) % 2]).wait()

        pltpu.sync_copy(dent, den_hbm.at[cid * NS + sid])
        plsc.subcore_barrier()
        base = cid * NPAD + sid * NPS
        pltpu.sync_copy(accs.at[pl.ds(sid * NPS, NPS)],
                        num_hbm.at[pl.ds(base, NPS)])

    return k


def _sc_edge_pass(xl, xr, src4, dst4, ea, consts, B):
    N, C = xl.shape
    NW, NGRP, NBS, _ = src4.shape
    k = _make_sc_edge_pass(N, C, NW, NGRP, NBS, B)
    return k(xl, xr, src4, dst4, ea, consts)


def kernel(x, edge_index, edge_attr,
           Wl1, bl1, Wr1, br1, We1, att1, b1,
           Wl2, bl2, Wr2, br2, We2, att2, b2):
    N, D = x.shape
    C = Wl1.shape[1]
    E = edge_index.shape[1]
    NW = SCC * NS
    EPT = E // NW
    B = 40
    assert E % NW == 0 and EPT % B == 0 and N % NS == 0

    NBS = next(n for n in range(10, 0, -1) if EPT % (n * B) == 0)
    NGRP = EPT // (NBS * B)
    assert NGRP * NBS * B == EPT
    src = edge_index[0].astype(jnp.int32)
    dst = edge_index[1].astype(jnp.int32)
    ea = edge_attr.reshape(E)
    src3 = src.reshape(NW, NGRP, NBS, B)
    dst3 = dst.reshape(NW, NGRP, NBS, B)
    consts1 = jnp.concatenate(
        [We1.reshape(1, C), att1.reshape(1, C)], axis=0)
    consts2 = jnp.concatenate(
        [We2.reshape(1, C), att2.reshape(1, C)], axis=0)

    xl1, xr1 = _tc_pre(x, Wl1, bl1, Wr1, br1)
    num1, den1 = _sc_edge_pass(xl1, xr1, src3, dst3, ea, consts1, B)
    xl2, xr2 = _tc_mid(num1, den1, b1, Wl2, bl2, Wr2, br2, N)
    num2, den2 = _sc_edge_pass(xl2, xr2, src3, dst3, ea, consts2, B)
    return _tc_post(num2, den2, b2, C, N)
